# baseline (speedup 1.0000x reference)
import functools

import jax
import jax.numpy as jnp
from jax import lax
from jax.experimental import pallas as pl
from jax.experimental.pallas import tpu as pltpu
from jax.experimental.pallas import tpu_sc as plsc

N = 10000
T = 12
FD = 256
HD = 256
NL = 3
E = 160000
OUT = 8

NC = 2
NS = 16
HALF = FD // NC
W = 128
NCH = 5
CW = 16
EP = NS * NCH * CW * W
PAD = EP - E
TRASH = 16
RB = 624
RW = 640

_mesh = plsc.VectorSubcoreMesh(core_axis_name="c", subcore_axis_name="s")


@functools.partial(
    pl.kernel,
    out_type=jax.ShapeDtypeStruct((NC, N, HALF), jnp.float32),
    mesh=_mesh,
    scratch_types=[
        pltpu.VMEM((CW, W), jnp.int32),
        pltpu.VMEM((CW, W), jnp.int32),
        pltpu.VMEM((W, HALF), jnp.float32),
        pltpu.VMEM((W, HALF), jnp.float32),
        pltpu.VMEM_SHARED((N + TRASH, HALF), jnp.float32),
        pltpu.SemaphoreType.DMA,
        pltpu.SemaphoreType.DMA,
    ],
)
def _agg(inp_hbm, src_hbm, dst_hbm, out_hbm, srcv, dstv, rows0, rows1,
         acc, sem0, sem1):
    c = lax.axis_index("c")
    s = lax.axis_index("s")

    def gath(w, buf, sem):
        return pltpu.async_copy(inp_hbm.at[c].at[srcv.at[w]], buf, sem)

    def scat(w, buf):
        pltpu.sync_copy(buf, acc.at[dstv.at[w]], add=True)

    pltpu.sync_copy(src_hbm.at[s].at[0], srcv)
    pltpu.sync_copy(dst_hbm.at[s].at[0], dstv)
    gath(0, rows0, sem0)

    @pl.loop(0, W)
    def _zrow(r):
        @pl.loop(0, HALF, step=16)
        def _zcol(j):
            rows1[r, pl.ds(j, 16)] = jnp.zeros((16,), jnp.float32)

    @pl.loop(0, RW, step=W)
    def _zacc(r0):
        pltpu.sync_copy(rows1, acc.at[pl.ds(s * RB + r0, W)])

    plsc.subcore_barrier()

    @pl.loop(0, NCH)
    def _chunk(kc):
        @pl.when(kc > 0)
        def _reload():
            pltpu.sync_copy(src_hbm.at[s].at[kc], srcv)
            pltpu.sync_copy(dst_hbm.at[s].at[kc], dstv)
            gath(0, rows0, sem0)

        @pl.loop(0, CW, step=2)
        def _win(w):
            gath(w + 1, rows1, sem1)
            pltpu.make_async_copy(inp_hbm.at[c].at[srcv.at[w]],
                                  rows0, sem0).wait()
            scat(w, rows0)

            @pl.when(w + 2 < CW)
            def _pref():
                gath(w + 2, rows0, sem0)

            pltpu.make_async_copy(inp_hbm.at[c].at[srcv.at[w + 1]],
                                  rows1, sem1).wait()
            scat(w + 1, rows1)

    plsc.subcore_barrier()
    pltpu.sync_copy(acc.at[pl.ds(s * RB, RW)],
                    out_hbm.at[c].at[pl.ds(s * RB, RW)])


def _make_xagg(nt, with_deg):
    out_slots = nt + (1 if with_deg else 0)

    @functools.partial(
        pl.kernel,
        out_type=jax.ShapeDtypeStruct((out_slots, NC, N, HALF), jnp.float32),
        mesh=_mesh,
        scratch_types=[
            pltpu.VMEM((CW, W), jnp.int32),
            pltpu.VMEM((CW, W), jnp.int32),
            pltpu.VMEM((W, HALF), jnp.float32),
            pltpu.VMEM((W, HALF), jnp.float32),
            pltpu.VMEM_SHARED((N + TRASH, HALF), jnp.float32),
            pltpu.SemaphoreType.DMA,
            pltpu.SemaphoreType.DMA,
        ],
    )
    def _xagg(xf_hbm, src_hbm, dst_hbm, out_hbm, srcv, dstv, rows0, rows1,
              acc, sem0, sem1):
        _xagg_body(nt, with_deg, xf_hbm, src_hbm, dst_hbm, out_hbm,
                   srcv, dstv, rows0, rows1, acc, sem0, sem1)

    return _xagg


def _xagg_body(nt, with_deg, xf_hbm, src_hbm, dst_hbm, out_hbm,
               srcv, dstv, rows0, rows1, acc, sem0, sem1):
    c = lax.axis_index("c")
    s = lax.axis_index("s")

    def fill_rows1(val):
        @pl.loop(0, W)
        def _fr(r):
            @pl.loop(0, HALF, step=16)
            def _fc_(j):
                rows1[r, pl.ds(j, 16)] = jnp.full((16,), val, jnp.float32)

    def zero_acc():
        @pl.loop(0, RW, step=W)
        def _za(r0):
            pltpu.sync_copy(rows1, acc.at[pl.ds(s * RB + r0, W)])

    def writeout(slot):
        pltpu.sync_copy(acc.at[pl.ds(s * RB, RW)],
                        out_hbm.at[slot].at[c].at[pl.ds(s * RB, RW)])

    if with_deg:
        fill_rows1(0.0)
        zero_acc()

        @pl.loop(0, W)
        def _or(r):
            @pl.loop(0, HALF, step=16)
            def _oc(j):
                rows0[r, pl.ds(j, 16)] = jnp.ones((16,), jnp.float32)

        plsc.subcore_barrier()

        @pl.loop(0, NCH)
        def _dchunk(kc):
            pltpu.sync_copy(dst_hbm.at[s].at[kc], dstv)

            @pl.loop(0, CW)
            def _dwin(w):
                pltpu.async_copy(rows0, acc.at[dstv.at[w]], sem0, add=True)

            @pl.loop(0, CW)
            def _ddrain(w):
                pltpu.make_async_copy(rows0, acc.at[dstv.at[w]],
                                      sem0).wait()

        plsc.subcore_barrier()
        writeout(nt)
        plsc.subcore_barrier()

    @pl.loop(0, nt)
    def _t(t):
        tc0 = t * NC + c
        pltpu.sync_copy(src_hbm.at[s].at[0], srcv)
        pltpu.sync_copy(dst_hbm.at[s].at[0], dstv)
        pltpu.async_copy(xf_hbm.at[tc0].at[srcv.at[0]], rows0, sem0)
        fill_rows1(0.0)
        zero_acc()
        plsc.subcore_barrier()

        @pl.loop(0, NCH)
        def _chunk(kc):
            @pl.when(kc > 0)
            def _reload():
                pltpu.sync_copy(src_hbm.at[s].at[kc], srcv)
                pltpu.sync_copy(dst_hbm.at[s].at[kc], dstv)
                pltpu.async_copy(xf_hbm.at[tc0].at[srcv.at[0]], rows0, sem0)

            @pl.loop(0, CW, step=2)
            def _win(w):
                pltpu.async_copy(xf_hbm.at[tc0].at[srcv.at[w + 1]],
                                 rows1, sem1)
                pltpu.make_async_copy(xf_hbm.at[tc0].at[srcv.at[w]],
                                      rows0, sem0).wait()
                pltpu.sync_copy(rows0, acc.at[dstv.at[w]], add=True)

                @pl.when(w + 2 < CW)
                def _pref():
                    pltpu.async_copy(xf_hbm.at[tc0].at[srcv.at[w + 2]],
                                     rows0, sem0)

                pltpu.make_async_copy(xf_hbm.at[tc0].at[srcv.at[w + 1]],
                                      rows1, sem1).wait()
                pltpu.sync_copy(rows1, acc.at[dstv.at[w + 1]], add=True)

        plsc.subcore_barrier()
        writeout(t)
        plsc.subcore_barrier()


_xagg_head = _make_xagg(1, True)
_xagg_tail = _make_xagg(T - 1, False)

MB = 2000
GRID = N // MB


def _dot(u, v):
    return lax.dot_general(u, v, (((1,), (0,)), ((), ())),
                           preferred_element_type=jnp.float32)


def _mm3(a, b_hi, b_lo):
    a_hi = a.astype(jnp.bfloat16)
    a_lo = (a - a_hi.astype(jnp.float32)).astype(jnp.bfloat16)
    return _dot(a_hi, b_hi) + _dot(a_hi, b_lo) + _dot(a_lo, b_hi)


def _cell_body(m_ref, h_ref, d_ref, whi_ref, wlo_ref, o_ref):
    deg = jnp.maximum(d_ref[...], 1.0)
    m = jnp.concatenate([m_ref[0], m_ref[1]], axis=1) / deg
    h = jnp.concatenate([h_ref[0], h_ref[1]], axis=1)
    g = _mm3(jnp.concatenate([m, h], axis=1), whi_ref[...], wlo_ref[...])
    z = jax.nn.sigmoid(g[:, :HD])
    r = jax.nn.sigmoid(g[:, HD:2 * HD])
    nn_ = jnp.tanh(g[:, 2 * HD:3 * HD] + r * g[:, 3 * HD:])
    hn = (1.0 - z) * nn_ + z * h
    o_ref[0] = hn[:, :HALF]
    o_ref[1] = hn[:, HALF:]


_cell = pl.pallas_call(
    _cell_body,
    grid=(GRID,),
    in_specs=[
        pl.BlockSpec((NC, MB, HALF), lambda i: (0, i, 0)),
        pl.BlockSpec((NC, MB, HALF), lambda i: (0, i, 0)),
        pl.BlockSpec((MB, 1), lambda i: (i, 0)),
        pl.BlockSpec((2 * HD, 4 * HD), lambda i: (0, 0)),
        pl.BlockSpec((2 * HD, 4 * HD), lambda i: (0, 0)),
    ],
    out_specs=pl.BlockSpec((NC, MB, HALF), lambda i: (0, i, 0)),
    out_shape=jax.ShapeDtypeStruct((NC, N, HALF), jnp.float32),
)


def _fc_body(h_ref, whi_ref, wlo_ref, b_ref, o_ref):
    h = jnp.concatenate([h_ref[0], h_ref[1]], axis=1)
    o_ref[...] = _mm3(h, whi_ref[...], wlo_ref[...]) + b_ref[...]


_fc = pl.pallas_call(
    _fc_body,
    grid=(GRID,),
    in_specs=[
        pl.BlockSpec((NC, MB, HALF), lambda i: (0, i, 0)),
        pl.BlockSpec((HD, OUT), lambda i: (0, 0)),
        pl.BlockSpec((HD, OUT), lambda i: (0, 0)),
        pl.BlockSpec((1, OUT), lambda i: (0, 0)),
    ],
    out_specs=pl.BlockSpec((MB, OUT), lambda i: (i, 0)),
    out_shape=jax.ShapeDtypeStruct((N, OUT), jnp.float32),
)


def kernel(x, edge_index, Wx, Wh, fc_W, fc_b):
    ar = jnp.arange(PAD, dtype=jnp.int32)
    src_pad = (ar * 2503) % N
    dst_pad = N + (ar % TRASH)
    src = jnp.concatenate([edge_index[0].astype(jnp.int32), src_pad])
    dst = jnp.concatenate([edge_index[1].astype(jnp.int32), dst_pad])
    src = src.reshape(NS, NCH, CW, W)
    dst = dst.reshape(NS, NCH, CW, W)
    xf = (jnp.moveaxis(x, 0, 1).reshape(T, N, NC, HALF)
          .transpose(0, 2, 1, 3).reshape(T * NC, N, HALF))

    head = _xagg_head(xf[:NC], src, dst)
    tail = _xagg_tail(xf[NC:], src, dst)
    deg = head[1, 0, :, 0:1]

    Wbig = jnp.zeros((NL, 2 * HD, 4 * HD), jnp.float32)
    Wbig = Wbig.at[:, :HD, :2 * HD].set(Wx[:, :, :2 * HD])
    Wbig = Wbig.at[:, HD:, :2 * HD].set(Wh[:, :, :2 * HD])
    Wbig = Wbig.at[:, :HD, 2 * HD:3 * HD].set(Wx[:, :, 2 * HD:])
    Wbig = Wbig.at[:, HD:, 3 * HD:].set(Wh[:, :, 2 * HD:])
    W_hi = Wbig.astype(jnp.bfloat16)
    W_lo = (Wbig - W_hi.astype(jnp.float32)).astype(jnp.bfloat16)
    fc_hi = fc_W.astype(jnp.bfloat16)
    fc_lo = (fc_W - fc_hi.astype(jnp.float32)).astype(jnp.bfloat16)

    hs = [jnp.zeros((NC, N, HALF), jnp.float32) for _ in range(NL)]
    for t in range(T):
        for l in range(NL):
            if l == 0:
                m = head[0] if t == 0 else tail[t - 1]
            else:
                m = _agg(hs[l - 1], src, dst)
            hs[l] = _cell(m, hs[l], deg, W_hi[l], W_lo[l])
    return _fc(hs[NL - 1], fc_hi, fc_lo, fc_b.reshape(1, OUT))


# speedup vs baseline: 1.0254x; 1.0254x over previous
import functools

import jax
import jax.numpy as jnp
from jax import lax
from jax.experimental import pallas as pl
from jax.experimental.pallas import tpu as pltpu
from jax.experimental.pallas import tpu_sc as plsc

N = 10000
T = 12
FD = 256
HD = 256
NL = 3
E = 160000
OUT = 8

NC = 2
NS = 16
HALF = FD // NC
W = 128
NCH = 4
CW = 20
EP = NS * NCH * CW * W
PAD = EP - E
TRASH = 16
RB = 624
RW = 640

_mesh = plsc.VectorSubcoreMesh(core_axis_name="c", subcore_axis_name="s")


@functools.partial(
    pl.kernel,
    out_type=jax.ShapeDtypeStruct((NC, N, HALF), jnp.float32),
    mesh=_mesh,
    scratch_types=[
        pltpu.VMEM((CW, W), jnp.int32),
        pltpu.VMEM((CW, W), jnp.int32),
        pltpu.VMEM((W, HALF), jnp.float32),
        pltpu.VMEM((W, HALF), jnp.float32),
        pltpu.VMEM_SHARED((N + TRASH, HALF), jnp.float32),
        pltpu.SemaphoreType.DMA,
        pltpu.SemaphoreType.DMA,
    ],
)
def _agg(inp_hbm, src_hbm, dst_hbm, out_hbm, srcv, dstv, rows0, rows1,
         acc, sem0, sem1):
    c = lax.axis_index("c")
    s = lax.axis_index("s")

    def gath(w, buf, sem):
        return pltpu.async_copy(inp_hbm.at[c].at[srcv.at[w]], buf, sem)

    def scat(w, buf):
        pltpu.sync_copy(buf, acc.at[dstv.at[w]], add=True)

    pltpu.sync_copy(src_hbm.at[s].at[0], srcv)
    pltpu.sync_copy(dst_hbm.at[s].at[0], dstv)
    gath(0, rows0, sem0)

    @pl.loop(0, W)
    def _zrow(r):
        @pl.loop(0, HALF, step=16)
        def _zcol(j):
            rows1[r, pl.ds(j, 16)] = jnp.zeros((16,), jnp.float32)

    @pl.loop(0, RW, step=W)
    def _zacc(r0):
        pltpu.sync_copy(rows1, acc.at[pl.ds(s * RB + r0, W)])

    plsc.subcore_barrier()

    @pl.loop(0, NCH)
    def _chunk(kc):
        @pl.when(kc > 0)
        def _reload():
            pltpu.sync_copy(src_hbm.at[s].at[kc], srcv)
            pltpu.sync_copy(dst_hbm.at[s].at[kc], dstv)
            gath(0, rows0, sem0)

        @pl.loop(0, CW, step=2)
        def _win(w):
            gath(w + 1, rows1, sem1)
            pltpu.make_async_copy(inp_hbm.at[c].at[srcv.at[w]],
                                  rows0, sem0).wait()
            scat(w, rows0)

            @pl.when(w + 2 < CW)
            def _pref():
                gath(w + 2, rows0, sem0)

            pltpu.make_async_copy(inp_hbm.at[c].at[srcv.at[w + 1]],
                                  rows1, sem1).wait()
            scat(w + 1, rows1)

    plsc.subcore_barrier()
    pltpu.sync_copy(acc.at[pl.ds(s * RB, RW)],
                    out_hbm.at[c].at[pl.ds(s * RB, RW)])


def _make_xagg(nt, with_deg):
    out_slots = nt + (1 if with_deg else 0)

    @functools.partial(
        pl.kernel,
        out_type=jax.ShapeDtypeStruct((out_slots, NC, N, HALF), jnp.float32),
        mesh=_mesh,
        scratch_types=[
            pltpu.VMEM((CW, W), jnp.int32),
            pltpu.VMEM((CW, W), jnp.int32),
            pltpu.VMEM((W, HALF), jnp.float32),
            pltpu.VMEM((W, HALF), jnp.float32),
            pltpu.VMEM_SHARED((N + TRASH, HALF), jnp.float32),
            pltpu.SemaphoreType.DMA,
            pltpu.SemaphoreType.DMA,
        ],
    )
    def _xagg(xf_hbm, src_hbm, dst_hbm, out_hbm, srcv, dstv, rows0, rows1,
              acc, sem0, sem1):
        _xagg_body(nt, with_deg, xf_hbm, src_hbm, dst_hbm, out_hbm,
                   srcv, dstv, rows0, rows1, acc, sem0, sem1)

    return _xagg


def _xagg_body(nt, with_deg, xf_hbm, src_hbm, dst_hbm, out_hbm,
               srcv, dstv, rows0, rows1, acc, sem0, sem1):
    c = lax.axis_index("c")
    s = lax.axis_index("s")

    def fill_rows1(val):
        @pl.loop(0, W)
        def _fr(r):
            @pl.loop(0, HALF, step=16)
            def _fc_(j):
                rows1[r, pl.ds(j, 16)] = jnp.full((16,), val, jnp.float32)

    def zero_acc():
        @pl.loop(0, RW, step=W)
        def _za(r0):
            pltpu.sync_copy(rows1, acc.at[pl.ds(s * RB + r0, W)])

    def writeout(slot):
        pltpu.sync_copy(acc.at[pl.ds(s * RB, RW)],
                        out_hbm.at[slot].at[c].at[pl.ds(s * RB, RW)])

    if with_deg:
        fill_rows1(0.0)
        zero_acc()

        @pl.loop(0, W)
        def _or(r):
            @pl.loop(0, HALF, step=16)
            def _oc(j):
                rows0[r, pl.ds(j, 16)] = jnp.ones((16,), jnp.float32)

        plsc.subcore_barrier()

        @pl.loop(0, NCH)
        def _dchunk(kc):
            pltpu.sync_copy(dst_hbm.at[s].at[kc], dstv)

            @pl.loop(0, CW)
            def _dwin(w):
                pltpu.async_copy(rows0, acc.at[dstv.at[w]], sem0, add=True)

            @pl.loop(0, CW)
            def _ddrain(w):
                pltpu.make_async_copy(rows0, acc.at[dstv.at[w]],
                                      sem0).wait()

        plsc.subcore_barrier()
        writeout(nt)
        plsc.subcore_barrier()

    @pl.loop(0, nt)
    def _t(t):
        tc0 = t * NC + c
        pltpu.sync_copy(src_hbm.at[s].at[0], srcv)
        pltpu.sync_copy(dst_hbm.at[s].at[0], dstv)
        pltpu.async_copy(xf_hbm.at[tc0].at[srcv.at[0]], rows0, sem0)
        fill_rows1(0.0)
        zero_acc()
        plsc.subcore_barrier()

        @pl.loop(0, NCH)
        def _chunk(kc):
            @pl.when(kc > 0)
            def _reload():
                pltpu.sync_copy(src_hbm.at[s].at[kc], srcv)
                pltpu.sync_copy(dst_hbm.at[s].at[kc], dstv)
                pltpu.async_copy(xf_hbm.at[tc0].at[srcv.at[0]], rows0, sem0)

            @pl.loop(0, CW, step=2)
            def _win(w):
                pltpu.async_copy(xf_hbm.at[tc0].at[srcv.at[w + 1]],
                                 rows1, sem1)
                pltpu.make_async_copy(xf_hbm.at[tc0].at[srcv.at[w]],
                                      rows0, sem0).wait()
                pltpu.sync_copy(rows0, acc.at[dstv.at[w]], add=True)

                @pl.when(w + 2 < CW)
                def _pref():
                    pltpu.async_copy(xf_hbm.at[tc0].at[srcv.at[w + 2]],
                                     rows0, sem0)

                pltpu.make_async_copy(xf_hbm.at[tc0].at[srcv.at[w + 1]],
                                      rows1, sem1).wait()
                pltpu.sync_copy(rows1, acc.at[dstv.at[w + 1]], add=True)

        plsc.subcore_barrier()
        writeout(t)
        plsc.subcore_barrier()


_xagg_head = _make_xagg(1, True)
_xagg_tail = _make_xagg(T - 1, False)

MB = 1000
GRID = N // MB


def _dot(u, v):
    return lax.dot_general(u, v, (((1,), (0,)), ((), ())),
                           preferred_element_type=jnp.float32)


def _mm3(a, b_hi, b_lo):
    a_hi = a.astype(jnp.bfloat16)
    a_lo = (a - a_hi.astype(jnp.float32)).astype(jnp.bfloat16)
    return _dot(a_hi, b_hi) + _dot(a_hi, b_lo) + _dot(a_lo, b_hi)


def _cell_body(m_ref, h_ref, d_ref, whi_ref, wlo_ref, o_ref):
    deg = jnp.maximum(d_ref[...], 1.0)
    m = jnp.concatenate([m_ref[0], m_ref[1]], axis=1) / deg
    h = jnp.concatenate([h_ref[0], h_ref[1]], axis=1)
    g = _mm3(jnp.concatenate([m, h], axis=1), whi_ref[...], wlo_ref[...])
    z = jax.nn.sigmoid(g[:, :HD])
    r = jax.nn.sigmoid(g[:, HD:2 * HD])
    nn_ = jnp.tanh(g[:, 2 * HD:3 * HD] + r * g[:, 3 * HD:])
    hn = (1.0 - z) * nn_ + z * h
    o_ref[0] = hn[:, :HALF]
    o_ref[1] = hn[:, HALF:]


_cell = pl.pallas_call(
    _cell_body,
    grid=(GRID,),
    in_specs=[
        pl.BlockSpec((NC, MB, HALF), lambda i: (0, i, 0)),
        pl.BlockSpec((NC, MB, HALF), lambda i: (0, i, 0)),
        pl.BlockSpec((MB, 1), lambda i: (i, 0)),
        pl.BlockSpec((2 * HD, 4 * HD), lambda i: (0, 0)),
        pl.BlockSpec((2 * HD, 4 * HD), lambda i: (0, 0)),
    ],
    out_specs=pl.BlockSpec((NC, MB, HALF), lambda i: (0, i, 0)),
    out_shape=jax.ShapeDtypeStruct((NC, N, HALF), jnp.float32),
)


def _fc_body(h_ref, whi_ref, wlo_ref, b_ref, o_ref):
    h = jnp.concatenate([h_ref[0], h_ref[1]], axis=1)
    o_ref[...] = _mm3(h, whi_ref[...], wlo_ref[...]) + b_ref[...]


_fc = pl.pallas_call(
    _fc_body,
    grid=(GRID,),
    in_specs=[
        pl.BlockSpec((NC, MB, HALF), lambda i: (0, i, 0)),
        pl.BlockSpec((HD, OUT), lambda i: (0, 0)),
        pl.BlockSpec((HD, OUT), lambda i: (0, 0)),
        pl.BlockSpec((1, OUT), lambda i: (0, 0)),
    ],
    out_specs=pl.BlockSpec((MB, OUT), lambda i: (i, 0)),
    out_shape=jax.ShapeDtypeStruct((N, OUT), jnp.float32),
)


def kernel(x, edge_index, Wx, Wh, fc_W, fc_b):
    ar = jnp.arange(PAD, dtype=jnp.int32)
    src_pad = (ar * 2503) % N
    dst_pad = N + (ar % TRASH)
    src = jnp.concatenate([edge_index[0].astype(jnp.int32), src_pad])
    dst = jnp.concatenate([edge_index[1].astype(jnp.int32), dst_pad])
    src = src.reshape(NS, NCH, CW, W)
    dst = dst.reshape(NS, NCH, CW, W)
    xf = (jnp.moveaxis(x, 0, 1).reshape(T, N, NC, HALF)
          .transpose(0, 2, 1, 3).reshape(T * NC, N, HALF))

    head = _xagg_head(xf[:NC], src, dst)
    tail = _xagg_tail(xf[NC:], src, dst)
    deg = head[1, 0, :, 0:1]

    Wbig = jnp.zeros((NL, 2 * HD, 4 * HD), jnp.float32)
    Wbig = Wbig.at[:, :HD, :2 * HD].set(Wx[:, :, :2 * HD])
    Wbig = Wbig.at[:, HD:, :2 * HD].set(Wh[:, :, :2 * HD])
    Wbig = Wbig.at[:, :HD, 2 * HD:3 * HD].set(Wx[:, :, 2 * HD:])
    Wbig = Wbig.at[:, HD:, 3 * HD:].set(Wh[:, :, 2 * HD:])
    W_hi = Wbig.astype(jnp.bfloat16)
    W_lo = (Wbig - W_hi.astype(jnp.float32)).astype(jnp.bfloat16)
    fc_hi = fc_W.astype(jnp.bfloat16)
    fc_lo = (fc_W - fc_hi.astype(jnp.float32)).astype(jnp.bfloat16)

    hs = [jnp.zeros((NC, N, HALF), jnp.float32) for _ in range(NL)]
    for t in range(T):
        for l in range(NL):
            if l == 0:
                m = head[0] if t == 0 else tail[t - 1]
            else:
                m = _agg(hs[l - 1], src, dst)
            hs[l] = _cell(m, hs[l], deg, W_hi[l], W_lo[l])
    return _fc(hs[NL - 1], fc_hi, fc_lo, fc_b.reshape(1, OUT))
